# baseline (speedup 1.0000x reference)
import functools

import jax
import jax.numpy as jnp
from jax import lax
from jax.experimental import pallas as pl
from jax.experimental.pallas import tpu as pltpu
from jax.experimental.pallas import tpu_sc as plsc

NC, NS = 2, 16
NW = NC * NS
B, L, D = 16384, 50, 32
NROWS = 1000001
BPW = B // NW
BR = 16
NBLK = BPW // BR
HALF = 16
INV_L = 1.0 / L

_mesh = plsc.VectorSubcoreMesh(
    core_axis_name="c", subcore_axis_name="s", num_cores=NC, num_subcores=NS
)


@functools.partial(
    pl.kernel,
    out_type=jax.ShapeDtypeStruct((B, D), jnp.float32),
    mesh=_mesh,
    scratch_types=[
        pltpu.VMEM((BPW, L), jnp.int32),
        pltpu.VMEM((2, BR, L, D), jnp.float32),
        pltpu.VMEM((BPW, D), jnp.float32),
        pltpu.SemaphoreType.DMA,
        pltpu.SemaphoreType.DMA,
    ],
    compiler_params=pltpu.CompilerParams(use_tc_tiling_on_sc=False),
)
def _emb_lookup_mean(table_hbm, idx_hbm, out_hbm, idx_v, rows_v, out_v,
                     sem0, sem1):
    sems = (sem0, sem1)
    wid = lax.axis_index("s") * NC + lax.axis_index("c")
    pltpu.sync_copy(idx_hbm.at[pl.ds(wid * BPW, BPW)], idx_v)

    def fire(p, blk):
        for r in range(BR):
            pltpu.async_copy(
                table_hbm.at[idx_v.at[blk * BR + r, :]],
                rows_v.at[p, r],
                sems[p],
            )

    def drain(p):
        for r in range(BR):
            pltpu.make_async_copy(
                table_hbm.at[idx_v.at[r, :]],
                rows_v.at[p, r],
                sems[p],
            ).wait()

    def accum(p, blk):
        def row(r, carry):
            acc0 = rows_v[p, r, 0, 0:HALF]
            acc1 = rows_v[p, r, 0, HALF:D]
            for l in range(1, L):
                acc0 = acc0 + rows_v[p, r, l, 0:HALF]
                acc1 = acc1 + rows_v[p, r, l, HALF:D]
            orow = blk * BR + r
            out_v[orow, 0:HALF] = acc0 * INV_L
            out_v[orow, HALF:D] = acc1 * INV_L
            return carry

        lax.fori_loop(0, BR, row, 0)

    fire(0, 0)

    def body(g2, carry):
        ga = 2 * g2
        fire(1, ga + 1)
        drain(0)
        accum(0, ga)
        fire(0, ga + 2)
        drain(1)
        accum(1, ga + 1)
        return carry

    lax.fori_loop(0, NBLK // 2 - 1, body, 0)

    fire(1, NBLK - 1)
    drain(0)
    accum(0, NBLK - 2)
    drain(1)
    accum(1, NBLK - 1)

    pltpu.sync_copy(out_v, out_hbm.at[pl.ds(wid * BPW, BPW)])


NTQ = 7812
TPW = NTQ // NW
NDG = 4
BLKF = 128 * D


@functools.partial(
    pl.kernel,
    out_type=jax.ShapeDtypeStruct((NROWS * D,), jnp.float32),
    mesh=_mesh,
    scratch_types=(
        [pltpu.VMEM((8, 128), jnp.float32) for _ in range(2 * NDG)]
        + [pltpu.VMEM((BLKF,), jnp.float32) for _ in range(2)]
        + [pltpu.SemaphoreType.DMA for _ in range(4)]
    ),
    compiler_params=pltpu.CompilerParams(
        use_tc_tiling_on_sc=True, needs_layout_passes=False
    ),
)
def _format_table(tt_hbm, out_hbm, *scr):
    in_refs = (scr[0:NDG], scr[NDG:2 * NDG])
    blk = (scr[2 * NDG], scr[2 * NDG + 1])
    sin = (scr[2 * NDG + 2], scr[2 * NDG + 3])
    sout = (scr[2 * NDG + 4], scr[2 * NDG + 5])
    wid = lax.axis_index("s") * NC + lax.axis_index("c")
    cnt = jnp.where(wid < 2, TPW + 2, TPW)
    q0 = TPW * wid + 2 * jnp.minimum(wid, 2)
    iota_d = lax.iota(jnp.int32, 16) * D

    def fire_in(p, q):
        col = pl.multiple_of(q * 128, 128)
        for dg in range(NDG):
            pltpu.async_copy(
                tt_hbm.at[pl.ds(dg * 8, 8), pl.ds(col, 128)],
                in_refs[p][dg],
                sin[p],
            )

    def wait_in(p):
        for dg in range(NDG):
            pltpu.make_async_copy(
                tt_hbm.at[pl.ds(dg * 8, 8), pl.ds(0, 128)],
                in_refs[p][dg],
                sin[p],
            ).wait()

    def shuffle(p):
        for dg in range(NDG):
            for d8 in range(8):
                d = dg * 8 + d8
                for cc in range(8):
                    vals = in_refs[p][dg][d8, pl.ds(cc * 16, 16)]
                    plsc.store_scatter(
                        blk[p], [iota_d + (cc * 16 * D + d)], vals
                    )

    def fire_out(p, q):
        off = pl.multiple_of(q * BLKF, 8)
        pltpu.async_copy(blk[p], out_hbm.at[pl.ds(off, BLKF)], sout[p])

    def wait_out(p):
        pltpu.make_async_copy(
            blk[p], out_hbm.at[pl.ds(0, BLKF)], sout[p]
        ).wait()

    fire_in(0, q0)
    fire_in(1, q0 + 1)
    wait_in(0)
    shuffle(0)
    fire_out(0, q0)
    fire_in(0, q0 + 2)
    wait_in(1)
    shuffle(1)
    fire_out(1, q0 + 1)

    def body(j, carry):
        qa = q0 + 2 * j
        fire_in(1, qa + 1)
        wait_in(0)
        wait_out(0)
        shuffle(0)
        fire_out(0, qa)
        fire_in(0, qa + 2)
        wait_in(1)
        wait_out(1)
        shuffle(1)
        fire_out(1, qa + 1)
        return carry

    lax.fori_loop(1, cnt // 2 - 1, body, 0)

    qa = q0 + cnt - 2
    fire_in(1, qa + 1)
    wait_in(0)
    wait_out(0)
    shuffle(0)
    fire_out(0, qa)
    wait_in(1)
    wait_out(1)
    shuffle(1)
    fire_out(1, qa + 1)
    wait_out(0)
    wait_out(1)


def kernel(inputs, table):
    flat = _format_table(table.T)
    tail = table[NTQ * 128:].reshape((NROWS - NTQ * 128) * D)
    flat = lax.dynamic_update_slice(flat, tail, (NTQ * 128 * D,))
    return _emb_lookup_mean(flat.reshape(NROWS, D), inputs)


# speedup vs baseline: 1.0418x; 1.0418x over previous
import functools

import jax
import jax.numpy as jnp
from jax import lax
from jax.experimental import pallas as pl
from jax.experimental.pallas import tpu as pltpu
from jax.experimental.pallas import tpu_sc as plsc

NC, NS = 2, 16
NW = NC * NS
B, L, D = 16384, 50, 32
NROWS = 1000001
BPW = B // NW
BR = 16
NBLK = BPW // BR
HALF = 16
INV_L = 1.0 / L

_mesh = plsc.VectorSubcoreMesh(
    core_axis_name="c", subcore_axis_name="s", num_cores=NC, num_subcores=NS
)


@functools.partial(
    pl.kernel,
    out_type=jax.ShapeDtypeStruct((B, D), jnp.float32),
    mesh=_mesh,
    scratch_types=[
        pltpu.VMEM((BPW, L), jnp.int32),
        pltpu.VMEM((2, BR, L, D), jnp.float32),
        pltpu.VMEM((BPW, D), jnp.float32),
        pltpu.SemaphoreType.DMA,
        pltpu.SemaphoreType.DMA,
    ],
    compiler_params=pltpu.CompilerParams(use_tc_tiling_on_sc=False),
)
def _emb_lookup_mean(table_hbm, idx_hbm, out_hbm, idx_v, rows_v, out_v,
                     sem0, sem1):
    sems = (sem0, sem1)
    wid = lax.axis_index("s") * NC + lax.axis_index("c")
    pltpu.sync_copy(idx_hbm.at[pl.ds(wid * BPW, BPW)], idx_v)

    def fire(p, blk):
        for r in range(BR):
            pltpu.async_copy(
                table_hbm.at[idx_v.at[blk * BR + r, :]],
                rows_v.at[p, r],
                sems[p],
            )

    def drain(p):
        for r in range(BR):
            pltpu.make_async_copy(
                table_hbm.at[idx_v.at[r, :]],
                rows_v.at[p, r],
                sems[p],
            ).wait()

    def accum(p, blk):
        def row(r, carry):
            acc0 = rows_v[p, r, 0, 0:HALF]
            acc1 = rows_v[p, r, 0, HALF:D]
            for l in range(1, L):
                acc0 = acc0 + rows_v[p, r, l, 0:HALF]
                acc1 = acc1 + rows_v[p, r, l, HALF:D]
            orow = blk * BR + r
            out_v[orow, 0:HALF] = acc0 * INV_L
            out_v[orow, HALF:D] = acc1 * INV_L
            return carry

        lax.fori_loop(0, BR, row, 0)

    fire(0, 0)

    def body(g2, carry):
        ga = 2 * g2
        fire(1, ga + 1)
        drain(0)
        accum(0, ga)
        fire(0, ga + 2)
        drain(1)
        accum(1, ga + 1)
        return carry

    lax.fori_loop(0, NBLK // 2 - 1, body, 0)

    fire(1, NBLK - 1)
    drain(0)
    accum(0, NBLK - 2)
    drain(1)
    accum(1, NBLK - 1)

    pltpu.sync_copy(out_v, out_hbm.at[pl.ds(wid * BPW, BPW)])


NTQ = 7812
TPW = NTQ // NW
NDG = 4
BLKF = 128 * D


@functools.partial(
    pl.kernel,
    out_type=jax.ShapeDtypeStruct((NROWS * D,), jnp.float32),
    mesh=_mesh,
    scratch_types=(
        [pltpu.VMEM((8, 128), jnp.float32) for _ in range(2 * NDG)]
        + [pltpu.VMEM((128 * 33,), jnp.float32) for _ in range(2)]
        + [pltpu.VMEM((BLKF,), jnp.float32) for _ in range(2)]
        + [pltpu.SemaphoreType.DMA for _ in range(4)]
    ),
    compiler_params=pltpu.CompilerParams(
        use_tc_tiling_on_sc=True, needs_layout_passes=False
    ),
)
def _format_table(tt_hbm, out_hbm, *scr):
    in_refs = (scr[0:NDG], scr[NDG:2 * NDG])
    blk2 = (scr[2 * NDG], scr[2 * NDG + 1])
    blk = (scr[2 * NDG + 2], scr[2 * NDG + 3])
    sin = (scr[2 * NDG + 4], scr[2 * NDG + 5])
    sout = (scr[2 * NDG + 6], scr[2 * NDG + 7])
    wid = lax.axis_index("s") * NC + lax.axis_index("c")
    cnt = jnp.where(wid < 2, TPW + 2, TPW)
    q0 = TPW * wid + 2 * jnp.minimum(wid, 2)
    iota = lax.iota(jnp.int32, 16)
    iota33 = iota * 33

    def fire_in(p, q):
        col = pl.multiple_of(q * 128, 128)
        for dg in range(NDG):
            pltpu.async_copy(
                tt_hbm.at[pl.ds(dg * 8, 8), pl.ds(col, 128)],
                in_refs[p][dg],
                sin[p],
            )

    def wait_in(p):
        for dg in range(NDG):
            pltpu.make_async_copy(
                tt_hbm.at[pl.ds(dg * 8, 8), pl.ds(0, 128)],
                in_refs[p][dg],
                sin[p],
            ).wait()

    def shuffle(p):
        for dg in range(NDG):
            for d8 in range(8):
                d = dg * 8 + d8
                for cc in range(8):
                    vals = in_refs[p][dg][d8, pl.ds(cc * 16, 16)]
                    plsc.store_scatter(
                        blk2[p], [iota33 + (cc * 16 * 33 + d)], vals
                    )
        for c in range(128):
            v0 = plsc.load_gather(blk2[p], [iota + (c * 33)])
            v1 = plsc.load_gather(blk2[p], [iota + (c * 33 + 16)])
            blk[p][pl.ds(c * D, 16)] = v0
            blk[p][pl.ds(c * D + 16, 16)] = v1

    def fire_out(p, q):
        off = pl.multiple_of(q * BLKF, 8)
        pltpu.async_copy(blk[p], out_hbm.at[pl.ds(off, BLKF)], sout[p])

    def wait_out(p):
        pltpu.make_async_copy(
            blk[p], out_hbm.at[pl.ds(0, BLKF)], sout[p]
        ).wait()

    fire_in(0, q0)
    fire_in(1, q0 + 1)

    def body(j, carry):
        qa = q0 + 2 * j
        for p in range(2):
            wait_in(p)

            @pl.when(j > 0)
            def _():
                wait_out(p)

            shuffle(p)
            fire_in(p, jnp.minimum(qa + 2 + p, NTQ - 1))
            fire_out(p, qa + p)
        return carry

    lax.fori_loop(0, cnt // 2, body, 0)

    wait_in(0)
    wait_in(1)
    wait_out(0)
    wait_out(1)


def kernel(inputs, table):
    flat = _format_table(table.T)
    tail = table[NTQ * 128:].reshape((NROWS - NTQ * 128) * D)
    flat = lax.dynamic_update_slice(flat, tail, (NTQ * 128 * D,))
    return _emb_lookup_mean(flat.reshape(NROWS, D), inputs)


# speedup vs baseline: 1.2394x; 1.1898x over previous
import functools

import jax
import jax.numpy as jnp
from jax import lax
from jax.experimental import pallas as pl
from jax.experimental.pallas import tpu as pltpu
from jax.experimental.pallas import tpu_sc as plsc

NC, NS = 2, 16
NW = NC * NS
B, L, D = 16384, 50, 32
NROWS = 1000001
BPW = B // NW
BR = 16
NBLK = BPW // BR
HALF = 16
INV_L = 1.0 / L

_mesh = plsc.VectorSubcoreMesh(
    core_axis_name="c", subcore_axis_name="s", num_cores=NC, num_subcores=NS
)


@functools.partial(
    pl.kernel,
    out_type=jax.ShapeDtypeStruct((B, D), jnp.float32),
    mesh=_mesh,
    scratch_types=[
        pltpu.VMEM((BPW, L), jnp.int32),
        pltpu.VMEM((2, BR, L, D), jnp.float32),
        pltpu.VMEM((BPW, D), jnp.float32),
        pltpu.SemaphoreType.DMA,
        pltpu.SemaphoreType.DMA,
    ],
    compiler_params=pltpu.CompilerParams(use_tc_tiling_on_sc=False),
)
def _emb_lookup_mean(table_hbm, idx_hbm, out_hbm, idx_v, rows_v, out_v,
                     sem0, sem1):
    sems = (sem0, sem1)
    wid = lax.axis_index("s") * NC + lax.axis_index("c")
    pltpu.sync_copy(idx_hbm.at[pl.ds(wid * BPW, BPW)], idx_v)

    def fire(p, blk):
        for r in range(BR):
            pltpu.async_copy(
                table_hbm.at[idx_v.at[blk * BR + r, :]],
                rows_v.at[p, r],
                sems[p],
            )

    def drain(p):
        for r in range(BR):
            pltpu.make_async_copy(
                table_hbm.at[idx_v.at[r, :]],
                rows_v.at[p, r],
                sems[p],
            ).wait()

    def accum(p, blk):
        def row(r, carry):
            acc0 = rows_v[p, r, 0, 0:HALF]
            acc1 = rows_v[p, r, 0, HALF:D]
            for l in range(1, L):
                acc0 = acc0 + rows_v[p, r, l, 0:HALF]
                acc1 = acc1 + rows_v[p, r, l, HALF:D]
            orow = blk * BR + r
            out_v[orow, 0:HALF] = acc0 * INV_L
            out_v[orow, HALF:D] = acc1 * INV_L
            return carry

        lax.fori_loop(0, BR, row, 0)

    fire(0, 0)

    def body(g2, carry):
        ga = 2 * g2
        fire(1, ga + 1)
        drain(0)
        accum(0, ga)
        fire(0, ga + 2)
        drain(1)
        accum(1, ga + 1)
        return carry

    lax.fori_loop(0, NBLK // 2 - 1, body, 0)

    fire(1, NBLK - 1)
    drain(0)
    accum(0, NBLK - 2)
    drain(1)
    accum(1, NBLK - 1)

    pltpu.sync_copy(out_v, out_hbm.at[pl.ds(wid * BPW, BPW)])


NTQ = 7812
TPW = NTQ // NW
NDG = 4
BLKF = 128 * D


@functools.partial(
    pl.kernel,
    out_type=jax.ShapeDtypeStruct((NROWS * D,), jnp.float32),
    mesh=_mesh,
    scratch_types=(
        [pltpu.VMEM((8, 128), jnp.float32) for _ in range(2 * NDG)]
        + [pltpu.VMEM((128 * 33,), jnp.float32) for _ in range(2)]
        + [pltpu.VMEM((BLKF,), jnp.float32) for _ in range(2)]
        + [pltpu.SemaphoreType.DMA for _ in range(4)]
    ),
    compiler_params=pltpu.CompilerParams(
        use_tc_tiling_on_sc=True, needs_layout_passes=False
    ),
)
def _format_table(tt_hbm, out_hbm, *scr):
    in_refs = (scr[0:NDG], scr[NDG:2 * NDG])
    blk2 = (scr[2 * NDG], scr[2 * NDG + 1])
    blk = (scr[2 * NDG + 2], scr[2 * NDG + 3])
    sin = (scr[2 * NDG + 4], scr[2 * NDG + 5])
    sout = (scr[2 * NDG + 6], scr[2 * NDG + 7])
    wid = lax.axis_index("s") * NC + lax.axis_index("c")
    cnt = jnp.where(wid < 2, TPW + 2, TPW)
    q0 = TPW * wid + 2 * jnp.minimum(wid, 2)
    iota = lax.iota(jnp.int32, 16)
    iota33 = iota * 33

    def fire_in(p, q):
        col = pl.multiple_of(q * 128, 128)
        for dg in range(NDG):
            pltpu.async_copy(
                tt_hbm.at[pl.ds(dg * 8, 8), pl.ds(col, 128)],
                in_refs[p][dg],
                sin[p],
            )

    def wait_in(p):
        for dg in range(NDG):
            pltpu.make_async_copy(
                tt_hbm.at[pl.ds(dg * 8, 8), pl.ds(0, 128)],
                in_refs[p][dg],
                sin[p],
            ).wait()

    def shuffle(p):
        for dg in range(NDG):
            for d8 in range(8):
                d = dg * 8 + d8
                for cc in range(8):
                    vals = in_refs[p][dg][d8, pl.ds(cc * 16, 16)]
                    plsc.store_scatter(
                        blk2[p], [iota33 + (cc * 16 * 33 + d)], vals
                    )
        for c0 in range(0, 128, 4):
            vs = []
            for c in range(c0, c0 + 4):
                vs.append(plsc.load_gather(blk2[p], [iota + (c * 33)]))
                vs.append(plsc.load_gather(blk2[p], [iota + (c * 33 + 16)]))
            for i, c in enumerate(range(c0, c0 + 4)):
                blk[p][pl.ds(c * D, 16)] = vs[2 * i]
                blk[p][pl.ds(c * D + 16, 16)] = vs[2 * i + 1]

    def fire_out(p, q):
        off = pl.multiple_of(q * BLKF, 8)
        pltpu.async_copy(blk[p], out_hbm.at[pl.ds(off, BLKF)], sout[p])

    def wait_out(p):
        pltpu.make_async_copy(
            blk[p], out_hbm.at[pl.ds(0, BLKF)], sout[p]
        ).wait()

    fire_in(0, q0)
    fire_in(1, q0 + 1)

    def body(j, carry):
        qa = q0 + 2 * j
        for p in range(2):
            wait_in(p)

            @pl.when(j > 0)
            def _():
                wait_out(p)

            shuffle(p)
            fire_in(p, jnp.minimum(qa + 2 + p, NTQ - 1))
            fire_out(p, qa + p)
        return carry

    lax.fori_loop(0, cnt // 2, body, 0)

    wait_in(0)
    wait_in(1)
    wait_out(0)
    wait_out(1)


def kernel(inputs, table):
    flat = _format_table(table.T)
    tail = table[NTQ * 128:].reshape((NROWS - NTQ * 128) * D)
    flat = lax.dynamic_update_slice(flat, tail, (NTQ * 128 * D,))
    return _emb_lookup_mean(flat.reshape(NROWS, D), inputs)


# speedup vs baseline: 1.3178x; 1.0632x over previous
import functools

import jax
import jax.numpy as jnp
from jax import lax
from jax.experimental import pallas as pl
from jax.experimental.pallas import tpu as pltpu
from jax.experimental.pallas import tpu_sc as plsc

NC, NS = 2, 16
NW = NC * NS
B, L, D = 16384, 50, 32
NROWS = 1000001
BPW = B // NW
BR = 16
NBLK = BPW // BR
HALF = 16
INV_L = 1.0 / L

_mesh = plsc.VectorSubcoreMesh(
    core_axis_name="c", subcore_axis_name="s", num_cores=NC, num_subcores=NS
)


@functools.partial(
    pl.kernel,
    out_type=jax.ShapeDtypeStruct((B, D), jnp.float32),
    mesh=_mesh,
    scratch_types=[
        pltpu.VMEM((BPW, L), jnp.int32),
        pltpu.VMEM((2, BR, L, D), jnp.float32),
        pltpu.VMEM((BPW, D), jnp.float32),
        pltpu.SemaphoreType.DMA,
        pltpu.SemaphoreType.DMA,
    ],
    compiler_params=pltpu.CompilerParams(use_tc_tiling_on_sc=False),
)
def _emb_lookup_mean(table_hbm, idx_hbm, out_hbm, idx_v, rows_v, out_v,
                     sem0, sem1):
    sems = (sem0, sem1)
    wid = lax.axis_index("s") * NC + lax.axis_index("c")
    pltpu.sync_copy(idx_hbm.at[pl.ds(wid * BPW, BPW)], idx_v)

    def fire(p, blk):
        for r in range(BR):
            pltpu.async_copy(
                table_hbm.at[idx_v.at[blk * BR + r, :]],
                rows_v.at[p, r],
                sems[p],
            )

    def drain(p):
        for r in range(BR):
            pltpu.make_async_copy(
                table_hbm.at[idx_v.at[r, :]],
                rows_v.at[p, r],
                sems[p],
            ).wait()

    def accum(p, blk):
        def row(r, carry):
            acc0 = rows_v[p, r, 0, 0:HALF]
            acc1 = rows_v[p, r, 0, HALF:D]
            for l in range(1, L):
                acc0 = acc0 + rows_v[p, r, l, 0:HALF]
                acc1 = acc1 + rows_v[p, r, l, HALF:D]
            orow = blk * BR + r
            out_v[orow, 0:HALF] = acc0 * INV_L
            out_v[orow, HALF:D] = acc1 * INV_L
            return carry

        lax.fori_loop(0, BR, row, 0)

    fire(0, 0)

    def body(g2, carry):
        ga = 2 * g2
        fire(1, ga + 1)
        drain(0)
        accum(0, ga)
        fire(0, ga + 2)
        drain(1)
        accum(1, ga + 1)
        return carry

    lax.fori_loop(0, NBLK // 2 - 1, body, 0)

    fire(1, NBLK - 1)
    drain(0)
    accum(0, NBLK - 2)
    drain(1)
    accum(1, NBLK - 1)

    pltpu.sync_copy(out_v, out_hbm.at[pl.ds(wid * BPW, BPW)])


NTQ = 7812
TPW = NTQ // NW
NDG = 4
BLKF = 128 * D


@functools.partial(
    pl.kernel,
    out_type=jax.ShapeDtypeStruct((NROWS * D,), jnp.float32),
    mesh=_mesh,
    scratch_types=(
        [pltpu.VMEM((8, 128), jnp.float32) for _ in range(2 * NDG)]
        + [pltpu.VMEM((128 * 33,), jnp.float32) for _ in range(2)]
        + [pltpu.VMEM((BLKF,), jnp.float32) for _ in range(2)]
        + [pltpu.SemaphoreType.DMA for _ in range(4)]
    ),
    compiler_params=pltpu.CompilerParams(
        use_tc_tiling_on_sc=True, needs_layout_passes=False
    ),
)
def _format_table(tt_hbm, out_hbm, *scr):
    in_refs = (scr[0:NDG], scr[NDG:2 * NDG])
    blk2 = (scr[2 * NDG], scr[2 * NDG + 1])
    blk = (scr[2 * NDG + 2], scr[2 * NDG + 3])
    sin = (scr[2 * NDG + 4], scr[2 * NDG + 5])
    sout = (scr[2 * NDG + 6], scr[2 * NDG + 7])
    wid = lax.axis_index("s") * NC + lax.axis_index("c")
    cnt = jnp.where(wid < 2, TPW + 2, TPW)
    q0 = TPW * wid + 2 * jnp.minimum(wid, 2)
    iota = lax.iota(jnp.int32, 16)
    iota33 = iota * 33

    def fire_in(p, q):
        col = pl.multiple_of(q * 128, 128)
        for dg in range(NDG):
            pltpu.async_copy(
                tt_hbm.at[pl.ds(dg * 8, 8), pl.ds(col, 128)],
                in_refs[p][dg],
                sin[p],
            )

    def wait_in(p):
        for dg in range(NDG):
            pltpu.make_async_copy(
                tt_hbm.at[pl.ds(dg * 8, 8), pl.ds(0, 128)],
                in_refs[p][dg],
                sin[p],
            ).wait()

    def shuffle(p):
        for dg in range(NDG):
            for d8 in range(8):
                d = dg * 8 + d8
                vals = [
                    in_refs[p][dg][d8, pl.ds(cc * 16, 16)] for cc in range(8)
                ]
                for cc in range(8):
                    plsc.store_scatter(
                        blk2[p], [iota33 + (cc * 16 * 33 + d)], vals[cc]
                    )
        for c0 in range(0, 128, 4):
            vs = []
            for c in range(c0, c0 + 4):
                vs.append(plsc.load_gather(blk2[p], [iota + (c * 33)]))
                vs.append(plsc.load_gather(blk2[p], [iota + (c * 33 + 16)]))
            for i, c in enumerate(range(c0, c0 + 4)):
                blk[p][pl.ds(c * D, 16)] = vs[2 * i]
                blk[p][pl.ds(c * D + 16, 16)] = vs[2 * i + 1]

    def fire_out(p, q):
        off = pl.multiple_of(q * BLKF, 8)
        pltpu.async_copy(blk[p], out_hbm.at[pl.ds(off, BLKF)], sout[p])

    def wait_out(p):
        pltpu.make_async_copy(
            blk[p], out_hbm.at[pl.ds(0, BLKF)], sout[p]
        ).wait()

    fire_in(0, q0)
    fire_in(1, q0 + 1)

    def body(j, carry):
        qa = q0 + 2 * j
        for p in range(2):
            wait_in(p)

            @pl.when(j > 0)
            def _():
                wait_out(p)

            shuffle(p)
            fire_in(p, jnp.minimum(qa + 2 + p, NTQ - 1))
            fire_out(p, qa + p)
        return carry

    lax.fori_loop(0, cnt // 2, body, 0)

    wait_in(0)
    wait_in(1)
    wait_out(0)
    wait_out(1)


def kernel(inputs, table):
    flat = _format_table(table.T)
    tail = table[NTQ * 128:].reshape((NROWS - NTQ * 128) * D)
    flat = lax.dynamic_update_slice(flat, tail, (NTQ * 128 * D,))
    return _emb_lookup_mean(flat.reshape(NROWS, D), inputs)
